# Initial kernel scaffold; baseline (speedup 1.0000x reference)
import jax
import jax.numpy as jnp
from jax.experimental import pallas as pl


def kernel(x, emb, Wk, bk, Wv, bv, Wg, bg, Wq, bq):
    raise NotImplementedError("write your pallas kernel here")


# speedup vs baseline: 1.5599x; 1.5599x over previous
import functools

import jax
import jax.numpy as jnp
from jax import lax
from jax.experimental import pallas as pl
from jax.experimental.pallas import tpu as pltpu
from jax.experimental.pallas import tpu_sc as plsc

_D = 768
_NM = 8
_NH = _NM + 1
_KV = _NH * _D
_C = 256


def _sc_gather(table, idx):
    info = plsc.get_sparse_core_info()
    num_workers = info.num_cores * info.num_subcores
    n = idx.shape[0]
    d = table.shape[1]
    per_w = n // num_workers
    mesh = plsc.VectorSubcoreMesh(core_axis_name="c", subcore_axis_name="s")

    @functools.partial(
        pl.kernel,
        mesh=mesh,
        out_type=jax.ShapeDtypeStruct((n, d), jnp.float32),
        scratch_types=[
            pltpu.VMEM((per_w,), jnp.int32),
            pltpu.VMEM((per_w, d), jnp.float32),
            pltpu.SemaphoreType.DMA,
        ],
    )
    def gather(table_hbm, idx_hbm, out_hbm, idx_v, rows_v, sem):
        wid = lax.axis_index("s") * info.num_cores + lax.axis_index("c")
        base = wid * per_w
        pltpu.sync_copy(idx_hbm.at[pl.ds(base, per_w)], idx_v)
        pltpu.async_copy(table_hbm.at[idx_v], rows_v, sem).wait()
        pltpu.sync_copy(rows_v, out_hbm.at[pl.ds(base, per_w)])

    return gather(table, idx)


def _proj_body(x_ref, wk_ref, wv_ref, wq_ref, bk_ref, bv_ref, bq_ref,
               ko_ref, vo_ref, qo_ref):
    j = pl.program_id(0)
    xb = x_ref[...]
    wk = wk_ref[...].astype(jnp.bfloat16)
    wv = wv_ref[...].astype(jnp.bfloat16)
    ko_ref[...] = (jnp.dot(xb, wk, preferred_element_type=jnp.float32)
                   + bk_ref[...]).astype(jnp.bfloat16)
    vo_ref[...] = (jnp.dot(xb, wv, preferred_element_type=jnp.float32)
                   + bv_ref[...]).astype(jnp.bfloat16)

    @pl.when(j == 0)
    def _():
        wq = wq_ref[...].astype(jnp.bfloat16)
        qo_ref[...] = (jnp.dot(xb, wq, preferred_element_type=jnp.float32)
                       + bq_ref[...]).astype(jnp.bfloat16)


def _projections(xe_bf, Wk, bk, Wv, bv, Wq, bq):
    s = xe_bf.shape[0]
    grid = (_NH,)
    out = pl.pallas_call(
        _proj_body,
        grid=grid,
        in_specs=[
            pl.BlockSpec((s, _D), lambda j: (0, 0)),
            pl.BlockSpec((_D, _D), lambda j: (0, j)),
            pl.BlockSpec((_D, _D), lambda j: (0, j)),
            pl.BlockSpec((_D, _D), lambda j: (0, 0)),
            pl.BlockSpec((1, _D), lambda j: (0, j)),
            pl.BlockSpec((1, _D), lambda j: (0, j)),
            pl.BlockSpec((1, _D), lambda j: (0, 0)),
        ],
        out_specs=[
            pl.BlockSpec((s, _D), lambda j: (0, j)),
            pl.BlockSpec((s, _D), lambda j: (0, j)),
            pl.BlockSpec((s, _D), lambda j: (0, 0)),
        ],
        out_shape=[
            jax.ShapeDtypeStruct((s, _KV), jnp.bfloat16),
            jax.ShapeDtypeStruct((s, _KV), jnp.bfloat16),
            jax.ShapeDtypeStruct((s, _D), jnp.bfloat16),
        ],
        compiler_params=pltpu.CompilerParams(
            dimension_semantics=("arbitrary",),
        ),
    )(xe_bf, Wk, Wv, Wq, bk[None, :], bv[None, :], bq[None, :])
    return out


def _attn_body(xe_ref, k_ref, v_ref, q_ref, wg_ref, bg_ref, o_ref, state_ref):
    t = pl.program_id(0)

    @pl.when(t == 0)
    def _():
        state_ref[...] = jnp.zeros_like(state_ref)

    xe = xe_ref[...]
    glog = jnp.dot(xe, wg_ref[...], preferred_element_type=jnp.float32) + bg_ref[...]
    lane = lax.broadcasted_iota(jnp.int32, (_C, 128), 1)
    glog = jnp.where(lane < _NM, glog, -1e30)
    v1 = jnp.max(glog, axis=1, keepdims=True)
    i1 = jnp.min(jnp.where(glog >= v1, lane, 128), axis=1, keepdims=True)
    sel1 = lane == i1
    g2 = jnp.where(sel1, -1e30, glog)
    v2 = jnp.max(g2, axis=1, keepdims=True)
    i2 = jnp.min(jnp.where(g2 >= v2, lane, 128), axis=1, keepdims=True)
    sel2 = lane == i2
    e = jnp.exp(v2 - v1)
    w1 = 1.0 / (1.0 + e)
    route = jnp.where(sel1, w1, 0.0) + jnp.where(sel2, 1.0 - w1, 0.0)

    q = q_ref[...]
    row = lax.broadcasted_iota(jnp.int32, (_C, _C), 0)
    col = lax.broadcasted_iota(jnp.int32, (_C, _C), 1)
    causal = col <= row

    o = jnp.zeros((_C, _D), jnp.float32)
    for m in range(_NH):
        km = k_ref[:, m * _D:(m + 1) * _D]
        vm = v_ref[:, m * _D:(m + 1) * _D]
        if m < _NM:
            gm = route[:, m:m + 1]
            kg = (km.astype(jnp.float32) * gm).astype(jnp.bfloat16)
        else:
            gm = None
            kg = km
        mb = state_ref[m * _D:(m + 1) * _D, :].astype(jnp.bfloat16)
        y = jnp.dot(q, mb, preferred_element_type=jnp.float32)
        s = lax.dot_general(q, kg, (((1,), (1,)), ((), ())),
                            preferred_element_type=jnp.float32)
        s = jnp.where(causal, s, 0.0).astype(jnp.bfloat16)
        y = y + jnp.dot(s, vm, preferred_element_type=jnp.float32)
        o = o + (gm * y if gm is not None else y)
        state_ref[m * _D:(m + 1) * _D, :] += lax.dot_general(
            kg, vm, (((0,), (0,)), ((), ())), preferred_element_type=jnp.float32)
    o_ref[...] = o


def _attention(xe, k, v, q, wg_pad, bg_pad):
    s = xe.shape[0]
    grid = (s // _C,)
    return pl.pallas_call(
        _attn_body,
        grid=grid,
        in_specs=[
            pl.BlockSpec((_C, _D), lambda t: (t, 0)),
            pl.BlockSpec((_C, _KV), lambda t: (t, 0)),
            pl.BlockSpec((_C, _KV), lambda t: (t, 0)),
            pl.BlockSpec((_C, _D), lambda t: (t, 0)),
            pl.BlockSpec((_D, 128), lambda t: (0, 0)),
            pl.BlockSpec((1, 128), lambda t: (0, 0)),
        ],
        out_specs=pl.BlockSpec((_C, _D), lambda t: (t, 0)),
        out_shape=jax.ShapeDtypeStruct((s, _D), jnp.float32),
        scratch_shapes=[pltpu.VMEM((_NH * _D, _D), jnp.float32)],
        compiler_params=pltpu.CompilerParams(
            dimension_semantics=("arbitrary",),
            fuse_transposed_lhs_in_matmul=True,
        ),
    )(xe, k, v, q, wg_pad, bg_pad)


def kernel(x, emb, Wk, bk, Wv, bv, Wg, bg, Wq, bq):
    b, s = x.shape
    idx = x.reshape(-1).astype(jnp.int32)
    xe = _sc_gather(emb, idx)
    xe_bf = xe.astype(jnp.bfloat16)
    k, v, q = _projections(xe_bf, Wk, bk, Wv, bv, Wq, bq)
    wg_pad = jnp.concatenate(
        [Wg, jnp.zeros((_D, 128 - _NM), jnp.float32)], axis=1)
    bg_pad = jnp.concatenate(
        [bg, jnp.zeros((128 - _NM,), jnp.float32)])[None, :]
    o = _attention(xe, k, v, q, wg_pad, bg_pad)
    return o.reshape(b, s, _D)
